# baseline (speedup 1.0000x reference)
import functools

import jax
import jax.numpy as jnp
from jax import lax
from jax.experimental import pallas as pl
from jax.experimental.pallas import tpu as pltpu
from jax.experimental.pallas import tpu_sc as plsc

NENT = 1000000
NREL = 1000
D = 64
B = 16384
GAMMA = 12.0

NW = 32
LANES = 16
B_PER_W = B // NW
CHUNK = 32
NCHUNK = B_PER_W // CHUNK
NGROUP = CHUNK // LANES


def _build_sc_call():
    mesh = plsc.VectorSubcoreMesh(core_axis_name="c", subcore_axis_name="s")

    @functools.partial(
        pl.kernel,
        mesh=mesh,
        out_type=jax.ShapeDtypeStruct((B,), jnp.float32),
        compiler_params=pltpu.CompilerParams(
            needs_layout_passes=False, use_tc_tiling_on_sc=True),
        scratch_types=[
            pltpu.VMEM((CHUNK,), jnp.int32),
            pltpu.VMEM((CHUNK,), jnp.int32),
            pltpu.VMEM((CHUNK,), jnp.int32),
            pltpu.VMEM((CHUNK * 8, D), jnp.float32),
            pltpu.VMEM((CHUNK * 8, D), jnp.float32),
            pltpu.VMEM((CHUNK, 2 * D), jnp.float32),
            pltpu.VMEM((CHUNK,), jnp.float32),
            pltpu.SemaphoreType.DMA,
            pltpu.SemaphoreType.DMA,
            pltpu.SemaphoreType.DMA,
        ],
    )
    def sc_pairre(head_hbm, rel_hbm, tail_hbm, ent_hbm, relemb_hbm, out_hbm,
                  hidx, tidx, ridx, hrows, trows, rrows, scores, s1, s2, s3):
        wid = lax.axis_index("s") * 2 + lax.axis_index("c")
        lane = lax.iota(jnp.int32, LANES)

        def chunk_body(c, _):
            base = pl.multiple_of(wid * B_PER_W + c * CHUNK, CHUNK)
            pltpu.sync_copy(head_hbm.at[pl.ds(base, CHUNK)], hidx)
            pltpu.sync_copy(tail_hbm.at[pl.ds(base, CHUNK)], tidx)
            pltpu.sync_copy(rel_hbm.at[pl.ds(base, CHUNK)], ridx)
            cr = pltpu.async_copy(relemb_hbm.at[ridx], rrows, s3)
            copies = []
            for k in range(CHUNK // LANES):
                hv = hidx[pl.ds(k * LANES, LANES)]
                tv = tidx[pl.ds(k * LANES, LANES)]
                for j in range(LANES):
                    i = k * LANES + j
                    hb = pl.multiple_of(hv[j] & ~7, 8)
                    tb = pl.multiple_of(tv[j] & ~7, 8)
                    copies.append(pltpu.async_copy(
                        ent_hbm.at[pl.ds(hb, 8), :],
                        hrows.at[pl.ds(i * 8, 8), :], s1))
                    copies.append(pltpu.async_copy(
                        ent_hbm.at[pl.ds(tb, 8), :],
                        trows.at[pl.ds(i * 8, 8), :], s2))
            for cp in copies:
                cp.wait()
            cr.wait()

            def group_body(g, _):
                row = g * LANES + lane
                hm = hidx[pl.ds(g * LANES, LANES)] & 7
                tm = tidx[pl.ds(g * LANES, LANES)] & 7
                hrow = row * 8 + hm
                trow = row * 8 + tm
                acc = jnp.zeros((LANES,), jnp.float32)
                for dd in range(D):
                    dv = jnp.full((LANES,), dd, jnp.int32)
                    vh = plsc.load_gather(hrows, [hrow, dv])
                    vt = plsc.load_gather(trows, [trow, dv])
                    vrh = plsc.load_gather(rrows, [row, dv])
                    vrt = plsc.load_gather(rrows, [row, dv + D])
                    acc = acc + jnp.abs(vh * vrh - vt * vrt)
                scores[pl.ds(g * LANES, LANES)] = GAMMA - acc
                return 0

            lax.fori_loop(0, NGROUP, group_body, 0)
            pltpu.sync_copy(scores, out_hbm.at[pl.ds(base, CHUNK)])
            return 0

        lax.fori_loop(0, NCHUNK, chunk_body, 0)

    return sc_pairre


def kernel(head, relation, tail, timestamps, entity_embedding, relation_embedding):
    del timestamps
    sc_pairre = _build_sc_call()
    out = sc_pairre(head.astype(jnp.int32), relation.astype(jnp.int32),
                    tail.astype(jnp.int32), entity_embedding, relation_embedding)
    return out.reshape(B, 1)


# speedup vs baseline: 1.3284x; 1.3284x over previous
import functools

import jax
import jax.numpy as jnp
from jax import lax
from jax.experimental import pallas as pl
from jax.experimental.pallas import tpu as pltpu
from jax.experimental.pallas import tpu_sc as plsc

NENT = 1000000
NREL = 1000
D = 64
B = 16384
GAMMA = 12.0

NW = 32
LANES = 16
B_PER_W = B // NW
CHUNK = 32
NCHUNK = B_PER_W // CHUNK
NGROUP = CHUNK // LANES


def _build_sc_call():
    mesh = plsc.VectorSubcoreMesh(core_axis_name="c", subcore_axis_name="s")

    @functools.partial(
        pl.kernel,
        mesh=mesh,
        out_type=jax.ShapeDtypeStruct((B,), jnp.float32),
        compiler_params=pltpu.CompilerParams(
            needs_layout_passes=False, use_tc_tiling_on_sc=True),
        scratch_types=[
            pltpu.VMEM((CHUNK,), jnp.int32),
            pltpu.VMEM((CHUNK,), jnp.int32),
            pltpu.VMEM((CHUNK,), jnp.int32),
            pltpu.VMEM((CHUNK * 8, D), jnp.float32),
            pltpu.VMEM((CHUNK * 8, D), jnp.float32),
            pltpu.VMEM((CHUNK, 2 * D), jnp.float32),
            pltpu.VMEM((CHUNK,), jnp.float32),
            pltpu.SemaphoreType.DMA,
            pltpu.SemaphoreType.DMA,
            pltpu.SemaphoreType.DMA,
        ],
    )
    def sc_pairre(head_hbm, rel_hbm, tail_hbm, ent_hbm, relemb_hbm, out_hbm,
                  hidx, tidx, ridx, hrows, trows, rrows, scores, s1, s2, s3):
        wid = lax.axis_index("s") * 2 + lax.axis_index("c")
        lane = lax.iota(jnp.int32, LANES)

        def chunk_body(c, _):
            base = pl.multiple_of(wid * B_PER_W + c * CHUNK, CHUNK)
            pltpu.sync_copy(head_hbm.at[pl.ds(base, CHUNK)], hidx)
            pltpu.sync_copy(tail_hbm.at[pl.ds(base, CHUNK)], tidx)
            pltpu.sync_copy(rel_hbm.at[pl.ds(base, CHUNK)], ridx)
            cr = pltpu.async_copy(relemb_hbm.at[ridx], rrows, s3)
            copies = []
            for k in range(CHUNK // LANES):
                hv = hidx[pl.ds(k * LANES, LANES)]
                tv = tidx[pl.ds(k * LANES, LANES)]
                for j in range(LANES):
                    i = k * LANES + j
                    hb = pl.multiple_of(hv[j] & ~7, 8)
                    tb = pl.multiple_of(tv[j] & ~7, 8)
                    copies.append(pltpu.async_copy(
                        ent_hbm.at[0, pl.ds(hb, 8), :],
                        hrows.at[pl.ds(i * 8, 8), :], s1))
                    copies.append(pltpu.async_copy(
                        ent_hbm.at[0, pl.ds(tb, 8), :],
                        trows.at[pl.ds(i * 8, 8), :], s2))
            for cp in copies:
                cp.wait()
            cr.wait()

            def group_body(g, _):
                row = g * LANES + lane
                hm = hidx[pl.ds(g * LANES, LANES)] & 7
                tm = tidx[pl.ds(g * LANES, LANES)] & 7
                hrow = row * 8 + hm
                trow = row * 8 + tm
                acc = jnp.zeros((LANES,), jnp.float32)
                for dd in range(D):
                    dv = jnp.full((LANES,), dd, jnp.int32)
                    vh = plsc.load_gather(hrows, [hrow, dv])
                    vt = plsc.load_gather(trows, [trow, dv])
                    vrh = plsc.load_gather(rrows, [row, dv])
                    vrt = plsc.load_gather(rrows, [row, dv + D])
                    acc = acc + jnp.abs(vh * vrh - vt * vrt)
                scores[pl.ds(g * LANES, LANES)] = GAMMA - acc
                return 0

            lax.fori_loop(0, NGROUP, group_body, 0)
            pltpu.sync_copy(scores, out_hbm.at[pl.ds(base, CHUNK)])
            return 0

        lax.fori_loop(0, NCHUNK, chunk_body, 0)

    return sc_pairre


def kernel(head, relation, tail, timestamps, entity_embedding, relation_embedding):
    del timestamps
    sc_pairre = _build_sc_call()
    ent3 = entity_embedding.reshape(1, NENT, D)
    out = sc_pairre(head.astype(jnp.int32), relation.astype(jnp.int32),
                    tail.astype(jnp.int32), ent3, relation_embedding)
    return out.reshape(B, 1)
